# baseline (speedup 1.0000x reference)
import jax
import jax.numpy as jnp
from jax.experimental import pallas as pl
from jax.experimental.pallas import tpu as pltpu

NF = 32
NB = 128
NT = 1024
NH = 16
NK = 5
ND = 4
G = 8
NGRP = NF // G
TC = 64
NC = NT // TC
E = TC + 4
CPS = (NT // ND) // TC
EB = E * NB
TB = TC * NB
ROWS = G * NH


def _dot(a, b):
    return jax.lax.dot_general(a, b, (((1,), (0,)), ((), ())),
                               preferred_element_type=jnp.float32)


def _kernel_body(x_ref, w1_ref, w2_ref, pw_ref, pb_ref, o_ref):
    xg = x_ref[0]
    w1 = w1_ref[0]
    w2a = w2_ref[0]
    ones8 = jnp.ones((G, EB), jnp.bfloat16)
    ones1 = jnp.ones((1, TB), jnp.bfloat16)
    segs = [None] * ND
    for c in range(NC):
        base = c * TC * NB
        xim = jnp.concatenate(
            [xg[:, base + k * NB: base + k * NB + EB] for k in range(NK)]
            + [ones8], axis=0)
        h1 = _dot(w1, xim)
        if c == 0:
            iot = jax.lax.broadcasted_iota(jnp.int32, (1, EB), 1)
            h1 = jnp.where(iot >= 2 * NB, h1, 0.0)
        if c == NC - 1:
            iot = jax.lax.broadcasted_iota(jnp.int32, (1, EB), 1)
            h1 = jnp.where(iot < (E - 2) * NB, h1, 0.0)
        h1b = jnp.maximum(h1.astype(jnp.bfloat16), 0)
        h5 = jnp.concatenate(
            [h1b[:, k * NB: k * NB + TB] for k in range(NK)] + [ones1],
            axis=0)
        h2 = jnp.maximum(_dot(w2a, h5), 0.0)
        r = h2
        w = TB
        while w > NB:
            w //= 2
            r = r[:, :w] + r[:, w: 2 * w]
        si = c // CPS
        segs[si] = r if segs[si] is None else segs[si] + r
    pmat = jnp.concatenate(segs, axis=0)
    o_ref[0] = _dot(pw_ref[0], pmat) + pb_ref[0]


def kernel(x, lengths, w1, b1, w2, b2, pw, pb):
    del lengths
    xt = jnp.transpose(x, (2, 1, 0))
    xp = jnp.pad(xt, ((0, 0), (4, 4), (0, 0)))
    x2 = xp.reshape(NGRP, G, (NT + 8) * NB).astype(jnp.bfloat16)

    eye = jnp.eye(G, dtype=jnp.float32)
    w1r = w1[:, :, 0, :].reshape(NGRP, G, NH, NK)
    w1bd = jnp.einsum('gfck,fe->gfcke', w1r, eye).reshape(NGRP, ROWS, NK * G)
    w1bd = jnp.concatenate(
        [w1bd, b1.reshape(NGRP, ROWS, 1),
         jnp.zeros((NGRP, ROWS, G - 1), jnp.float32)], axis=2)
    w1bd = w1bd.astype(jnp.bfloat16)
    w2r = w2.reshape(NGRP, G, NH, NH, NK)
    w2bd = jnp.einsum('gfoik,fe->gkfoei', w2r, eye).reshape(NGRP, NK, ROWS, ROWS)
    w2all = jnp.concatenate(
        [w2bd[:, k] for k in range(NK)] + [b2.reshape(NGRP, ROWS, 1)],
        axis=2).astype(jnp.bfloat16)
    pwr = (pw / (NT // ND)).reshape(NGRP, G, ND, NH, ND)
    pwbd = jnp.einsum('gfoid,fe->gfodei', pwr, eye)
    pwbd = pwbd.reshape(NGRP, G * ND, ND * ROWS)
    pbg = pb.reshape(NGRP, G * ND, 1)

    out = pl.pallas_call(
        _kernel_body,
        grid=(NGRP,),
        in_specs=[
            pl.BlockSpec((1, G, (NT + 8) * NB), lambda g: (g, 0, 0)),
            pl.BlockSpec((1, ROWS, NK * G + G), lambda g: (g, 0, 0)),
            pl.BlockSpec((1, ROWS, NK * ROWS + 1), lambda g: (g, 0, 0)),
            pl.BlockSpec((1, G * ND, ND * ROWS), lambda g: (g, 0, 0)),
            pl.BlockSpec((1, G * ND, 1), lambda g: (g, 0, 0)),
        ],
        out_specs=pl.BlockSpec((1, G * ND, NB), lambda g: (g, 0, 0)),
        out_shape=jax.ShapeDtypeStruct((NGRP, G * ND, NB), jnp.float32),
        compiler_params=pltpu.CompilerParams(
            dimension_semantics=("parallel",),
            vmem_limit_bytes=50 * 1024 * 1024,
        ),
    )(x2, w1bd, w2all, pwbd, pbg)

    return out.reshape(NF * ND, NB).T


# speedup vs baseline: 1.0668x; 1.0668x over previous
import jax
import jax.numpy as jnp
from jax.experimental import pallas as pl
from jax.experimental.pallas import tpu as pltpu

NF = 32
NB = 128
NT = 1024
NH = 16
NK = 5
ND = 4
G = 8
NGRP = NF // G
TC = 64
NC = NT // TC
E = TC + 4
CPS = (NT // ND) // TC
EB = E * NB
TB = TC * NB
ROWS = G * NH


def _dot(a, b):
    return jax.lax.dot_general(a, b, (((1,), (0,)), ((), ())),
                               preferred_element_type=jnp.float32)


def _xslice(xg, c, k):
    start = c * TC + k - 4
    if start < 0:
        m = -start
        return jnp.concatenate(
            [jnp.zeros((G, m * NB), jnp.bfloat16),
             xg[:, : (E - m) * NB]], axis=1)
    if start + E > NT:
        m = start + E - NT
        return jnp.concatenate(
            [xg[:, start * NB:],
             jnp.zeros((G, m * NB), jnp.bfloat16)], axis=1)
    return xg[:, start * NB: (start + E) * NB]


def _kernel_body(x_ref, w1_ref, w2_ref, pw_ref, pb_ref, o_ref):
    xg = x_ref[0]
    w1 = w1_ref[0]
    w2a = w2_ref[0]
    ones8 = jnp.ones((G, EB), jnp.bfloat16)
    ones1 = jnp.ones((1, TB), jnp.bfloat16)
    segs = [None] * ND
    for c in range(NC):
        xim = jnp.concatenate(
            [_xslice(xg, c, k) for k in range(NK)]
            + [ones8], axis=0)
        h1 = _dot(w1, xim)
        if c == 0:
            iot = jax.lax.broadcasted_iota(jnp.int32, (1, EB), 1)
            h1 = jnp.where(iot >= 2 * NB, h1, 0.0)
        if c == NC - 1:
            iot = jax.lax.broadcasted_iota(jnp.int32, (1, EB), 1)
            h1 = jnp.where(iot < (E - 2) * NB, h1, 0.0)
        h1b = jnp.maximum(h1.astype(jnp.bfloat16), 0)
        h5 = jnp.concatenate(
            [h1b[:, k * NB: k * NB + TB] for k in range(NK)] + [ones1],
            axis=0)
        h2 = jnp.maximum(_dot(w2a, h5), 0.0)
        r = h2
        w = TB
        while w > NB:
            w //= 2
            r = r[:, :w] + r[:, w: 2 * w]
        si = c // CPS
        segs[si] = r if segs[si] is None else segs[si] + r
    pmat = jnp.concatenate(segs, axis=0)
    o_ref[0] = _dot(pw_ref[0], pmat) + pb_ref[0]


def kernel(x, lengths, w1, b1, w2, b2, pw, pb):
    del lengths
    xt = jnp.transpose(x.astype(jnp.bfloat16), (2, 1, 0))
    x2 = xt.reshape(NGRP, G, NT * NB)

    eye = jnp.eye(G, dtype=jnp.float32)
    w1r = w1[:, :, 0, :].reshape(NGRP, G, NH, NK)
    w1bd = jnp.einsum('gfck,fe->gfcke', w1r, eye).reshape(NGRP, ROWS, NK * G)
    w1bd = jnp.concatenate(
        [w1bd, b1.reshape(NGRP, ROWS, 1),
         jnp.zeros((NGRP, ROWS, G - 1), jnp.float32)], axis=2)
    w1bd = w1bd.astype(jnp.bfloat16)
    w2r = w2.reshape(NGRP, G, NH, NH, NK)
    w2bd = jnp.einsum('gfoik,fe->gkfoei', w2r, eye).reshape(NGRP, NK, ROWS, ROWS)
    w2all = jnp.concatenate(
        [w2bd[:, k] for k in range(NK)] + [b2.reshape(NGRP, ROWS, 1)],
        axis=2).astype(jnp.bfloat16)
    pwr = (pw / (NT // ND)).reshape(NGRP, G, ND, NH, ND)
    pwbd = jnp.einsum('gfoid,fe->gfodei', pwr, eye)
    pwbd = pwbd.reshape(NGRP, G * ND, ND * ROWS)
    pbg = pb.reshape(NGRP, G * ND, 1)

    out = pl.pallas_call(
        _kernel_body,
        grid=(NGRP,),
        in_specs=[
            pl.BlockSpec((1, G, NT * NB), lambda g: (g, 0, 0)),
            pl.BlockSpec((1, ROWS, NK * G + G), lambda g: (g, 0, 0)),
            pl.BlockSpec((1, ROWS, NK * ROWS + 1), lambda g: (g, 0, 0)),
            pl.BlockSpec((1, G * ND, ND * ROWS), lambda g: (g, 0, 0)),
            pl.BlockSpec((1, G * ND, 1), lambda g: (g, 0, 0)),
        ],
        out_specs=pl.BlockSpec((1, G * ND, NB), lambda g: (g, 0, 0)),
        out_shape=jax.ShapeDtypeStruct((NGRP, G * ND, NB), jnp.float32),
        compiler_params=pltpu.CompilerParams(
            dimension_semantics=("parallel",),
            vmem_limit_bytes=50 * 1024 * 1024,
        ),
    )(x2, w1bd, w2all, pwbd, pbg)

    return out.reshape(NF * ND, NB).T


# speedup vs baseline: 1.1361x; 1.0649x over previous
import jax
import jax.numpy as jnp
from jax.experimental import pallas as pl
from jax.experimental.pallas import tpu as pltpu

NF = 32
NB = 128
NT = 1024
NH = 16
NK = 5
ND = 4
G = 8
NGRP = NF // G
TC = 64
NC = NT // TC
E = TC + 4
CPS = (NT // ND) // TC
EB = E * NB
TB = TC * NB
ROWS = G * NH


def _dot(a, b):
    return jax.lax.dot_general(a, b, (((1,), (0,)), ((), ())),
                               preferred_element_type=jnp.float32)


def _xslice(xg, c, k):
    start = c * TC + k - 4
    if start < 0:
        m = -start
        return jnp.concatenate(
            [jnp.zeros((G, m * NB), jnp.bfloat16),
             xg[:, : (E - m) * NB]], axis=1)
    if start + E > NT:
        m = start + E - NT
        return jnp.concatenate(
            [xg[:, start * NB:],
             jnp.zeros((G, m * NB), jnp.bfloat16)], axis=1)
    return xg[:, start * NB: (start + E) * NB]


def _compute_group(xg, w1, w2a, pw):
    ones8 = jnp.ones((G, EB), jnp.bfloat16)
    ones1 = jnp.ones((1, TB), jnp.bfloat16)
    segs = [None] * ND
    for c in range(NC):
        xim = jnp.concatenate(
            [_xslice(xg, c, k) for k in range(NK)]
            + [ones8], axis=0)
        h1 = _dot(w1, xim)
        if c == 0:
            iot = jax.lax.broadcasted_iota(jnp.int32, (1, EB), 1)
            h1 = jnp.where(iot >= 2 * NB, h1, 0.0)
        if c == NC - 1:
            iot = jax.lax.broadcasted_iota(jnp.int32, (1, EB), 1)
            h1 = jnp.where(iot < (E - 2) * NB, h1, 0.0)
        h1b = jnp.maximum(h1.astype(jnp.bfloat16), 0)
        h5 = jnp.concatenate(
            [h1b[:, k * NB: k * NB + TB] for k in range(NK)] + [ones1],
            axis=0)
        h2 = jnp.maximum(_dot(w2a, h5), 0.0)
        r = h2
        w = TB
        while w > NB:
            w //= 2
            r = r[:, :w] + r[:, w: 2 * w]
        si = c // CPS
        segs[si] = r if segs[si] is None else segs[si] + r
    pmat = jnp.concatenate(
        segs + [jnp.ones((1, NB), jnp.float32)], axis=0)
    return _dot(pw, pmat)


def kernel(x, lengths, w1, b1, w2, b2, pw, pb):
    del lengths
    xt = jnp.transpose(x.astype(jnp.bfloat16), (2, 1, 0))
    x2 = xt.reshape(NGRP, G, NT * NB)

    eye = jnp.eye(G, dtype=jnp.float32)
    w1r = w1[:, :, 0, :].reshape(NGRP, G, NH, NK)
    w1bd = jnp.einsum('gfck,fe->gfcke', w1r, eye).reshape(NGRP, ROWS, NK * G)
    w1bd = jnp.concatenate(
        [w1bd, b1.reshape(NGRP, ROWS, 1),
         jnp.zeros((NGRP, ROWS, G - 1), jnp.float32)], axis=2)
    w1bd = w1bd.astype(jnp.bfloat16)
    w2r = w2.reshape(NGRP, G, NH, NH, NK)
    w2bd = jnp.einsum('gfoik,fe->gkfoei', w2r, eye).reshape(NGRP, NK, ROWS, ROWS)
    w2all = jnp.concatenate(
        [w2bd[:, k] for k in range(NK)] + [b2.reshape(NGRP, ROWS, 1)],
        axis=2).astype(jnp.bfloat16)
    pwr = (pw / (NT // ND)).reshape(NGRP, G, ND, NH, ND)
    pwbd = jnp.einsum('gfoid,fe->gfodei', pwr, eye)
    pwbd = pwbd.reshape(NGRP, G * ND, ND * ROWS)
    pwbd = jnp.concatenate([pwbd, pb.reshape(NGRP, G * ND, 1)], axis=2)

    mesh = pltpu.create_tensorcore_mesh("core", num_cores=2)
    gpc = NGRP // 2

    def run(refs):
        x_hbm, w1_hbm, w2_hbm, pw_hbm, o_hbm = refs

        @pl.core_map(mesh,
                     compiler_params=pltpu.CompilerParams(
                         vmem_limit_bytes=50 * 1024 * 1024))
        def _():
            core = jax.lax.axis_index("core")

            def scoped(xbuf, w1buf, w2buf, pwbuf, obuf, sems):
                for gl in range(gpc):
                    g = core * gpc + gl
                    cps = [pltpu.make_async_copy(x_hbm.at[g], xbuf, sems.at[0]),
                           pltpu.make_async_copy(w1_hbm.at[g], w1buf, sems.at[1]),
                           pltpu.make_async_copy(w2_hbm.at[g], w2buf, sems.at[2]),
                           pltpu.make_async_copy(pw_hbm.at[g], pwbuf, sems.at[3])]
                    for cp in cps:
                        cp.start()
                    for cp in cps:
                        cp.wait()
                    obuf[...] = _compute_group(
                        xbuf[...], w1buf[...], w2buf[...], pwbuf[...])
                    ocp = pltpu.make_async_copy(obuf, o_hbm.at[g], sems.at[4])
                    ocp.start()
                    ocp.wait()

            pl.run_scoped(
                scoped,
                pltpu.VMEM((G, NT * NB), jnp.bfloat16),
                pltpu.VMEM((ROWS, NK * G + G), jnp.bfloat16),
                pltpu.VMEM((ROWS, NK * ROWS + 1), jnp.bfloat16),
                pltpu.VMEM((G * ND, ND * ROWS + 1), jnp.float32),
                pltpu.VMEM((G * ND, NB), jnp.float32),
                pltpu.SemaphoreType.DMA((5,)),
            )

    out0 = jnp.zeros((NGRP, G * ND, NB), jnp.float32)
    _, _, _, _, out = pl.run_state(run)((x2, w1bd, w2all, pwbd, out0))
    return out.reshape(NF * ND, NB).T


# speedup vs baseline: 1.1439x; 1.0069x over previous
import jax
import jax.numpy as jnp
from jax.experimental import pallas as pl
from jax.experimental.pallas import tpu as pltpu

NF = 32
NB = 128
NT = 1024
NH = 16
NK = 5
ND = 4
G = 8
NGRP = NF // G
TC = 64
NC = NT // TC
E = TC + 4
CPS = (NT // ND) // TC
EB = E * NB
TB = TC * NB
ROWS = G * NH


def _dot(a, b):
    return jax.lax.dot_general(a, b, (((1,), (0,)), ((), ())),
                               preferred_element_type=jnp.float32)


def _xslice(xg, c, k):
    start = c * TC + k - 4
    if start < 0:
        m = -start
        return jnp.concatenate(
            [jnp.zeros((G, m * NB), jnp.bfloat16),
             xg[:, : (E - m) * NB]], axis=1)
    if start + E > NT:
        m = start + E - NT
        return jnp.concatenate(
            [xg[:, start * NB:],
             jnp.zeros((G, m * NB), jnp.bfloat16)], axis=1)
    return xg[:, start * NB: (start + E) * NB]


def _compute_group(xg, w1, w2a, pw):
    ones8 = jnp.ones((G, EB), jnp.bfloat16)
    ones1 = jnp.ones((1, TB), jnp.bfloat16)
    segs = [None] * ND
    for c in range(NC):
        xim = jnp.concatenate(
            [_xslice(xg, c, k) for k in range(NK)]
            + [ones8], axis=0)
        h1 = _dot(w1, xim)
        if c == 0:
            iot = jax.lax.broadcasted_iota(jnp.int32, (1, EB), 1)
            h1 = jnp.where(iot >= 2 * NB, h1, 0.0)
        if c == NC - 1:
            iot = jax.lax.broadcasted_iota(jnp.int32, (1, EB), 1)
            h1 = jnp.where(iot < (E - 2) * NB, h1, 0.0)
        h1b = jnp.maximum(h1.astype(jnp.bfloat16), 0)
        h5 = jnp.concatenate(
            [h1b[:, k * NB: k * NB + TB] for k in range(NK)] + [ones1],
            axis=0)
        h2 = jnp.maximum(_dot(w2a, h5), 0.0)
        r = h2
        w = TB
        while w > NB:
            w //= 2
            r = r[:, :w] + r[:, w: 2 * w]
        si = c // CPS
        segs[si] = r if segs[si] is None else segs[si] + r
    pmat = jnp.concatenate(
        segs + [jnp.ones((1, NB), jnp.float32)], axis=0)
    return _dot(pw, pmat)


def kernel(x, lengths, w1, b1, w2, b2, pw, pb):
    del lengths
    xt = jnp.transpose(x.astype(jnp.bfloat16), (2, 1, 0))
    x2 = xt.reshape(NGRP, G, NT * NB)

    eye = jnp.eye(G, dtype=jnp.float32)
    w1r = w1[:, :, 0, :].reshape(NGRP, G, NH, NK)
    w1bd = jnp.einsum('gfck,fe->gfcke', w1r, eye).reshape(NGRP, ROWS, NK * G)
    w1bd = jnp.concatenate(
        [w1bd, b1.reshape(NGRP, ROWS, 1),
         jnp.zeros((NGRP, ROWS, G - 1), jnp.float32)], axis=2)
    w1bd = w1bd.astype(jnp.bfloat16)
    w2r = w2.reshape(NGRP, G, NH, NH, NK)
    w2bd = jnp.einsum('gfoik,fe->gkfoei', w2r, eye).reshape(NGRP, NK, ROWS, ROWS)
    w2all = jnp.concatenate(
        [w2bd[:, k] for k in range(NK)] + [b2.reshape(NGRP, ROWS, 1)],
        axis=2).astype(jnp.bfloat16)
    pwr = (pw / (NT // ND)).reshape(NGRP, G, ND, NH, ND)
    pwbd = jnp.einsum('gfoid,fe->gfodei', pwr, eye)
    pwbd = pwbd.reshape(NGRP, G * ND, ND * ROWS)
    pwbd = jnp.concatenate([pwbd, pb.reshape(NGRP, G * ND, 1)], axis=2)

    mesh = pltpu.create_tensorcore_mesh("core", num_cores=2)
    gpc = NGRP // 2

    def run(refs):
        x_hbm, w1_hbm, w2_hbm, pw_hbm, o_hbm = refs

        @pl.core_map(mesh,
                     compiler_params=pltpu.CompilerParams(
                         vmem_limit_bytes=30 * 1024 * 1024))
        def _():
            core = jax.lax.axis_index("core")

            def scoped(xbuf, w1buf, w2buf, pwbuf, obuf, sems):
                for gl in range(gpc):
                    g = core * gpc + gl
                    cps = [pltpu.make_async_copy(x_hbm.at[g], xbuf, sems.at[0]),
                           pltpu.make_async_copy(w1_hbm.at[g], w1buf, sems.at[1]),
                           pltpu.make_async_copy(w2_hbm.at[g], w2buf, sems.at[2]),
                           pltpu.make_async_copy(pw_hbm.at[g], pwbuf, sems.at[3])]
                    for cp in cps:
                        cp.start()
                    for cp in cps:
                        cp.wait()
                    obuf[...] = _compute_group(
                        xbuf[...], w1buf[...], w2buf[...], pwbuf[...])
                    ocp = pltpu.make_async_copy(obuf, o_hbm.at[g], sems.at[4])
                    ocp.start()
                    ocp.wait()

            pl.run_scoped(
                scoped,
                pltpu.VMEM((G, NT * NB), jnp.bfloat16),
                pltpu.VMEM((ROWS, NK * G + G), jnp.bfloat16),
                pltpu.VMEM((ROWS, NK * ROWS + 1), jnp.bfloat16),
                pltpu.VMEM((G * ND, ND * ROWS + 1), jnp.float32),
                pltpu.VMEM((G * ND, NB), jnp.float32),
                pltpu.SemaphoreType.DMA((5,)),
            )

    out0 = jnp.zeros((NGRP, G * ND, NB), jnp.float32)
    _, _, _, _, out = pl.run_state(run)((x2, w1bd, w2all, pwbd, out0))
    return out.reshape(NF * ND, NB).T
